# Initial kernel scaffold; baseline (speedup 1.0000x reference)
import jax
import jax.numpy as jnp
from jax.experimental import pallas as pl


def kernel(user_weight, item_weight, edge_index):
    raise NotImplementedError("write your pallas kernel here")


# speedup vs baseline: 7.1799x; 7.1799x over previous
import functools

import jax
import jax.numpy as jnp
from jax import lax
from jax.experimental import pallas as pl
from jax.experimental.pallas import tpu as pltpu, tpu_sc as plsc

N_USERS = 10000
N_ITEMS = 40000
N = N_USERS + N_ITEMS
N_PAD = 51200
RPT = N_PAD // 16
CPY = 25
HALF = 32
E = 800000
K = 128
E_PAD = 802816
NROW2D = E_PAD // K
C32 = NROW2D // 32
C16 = NROW2D // 16
TRASH = N_PAD - 1
BLK = RPT

_mesh = plsc.VectorSubcoreMesh(core_axis_name="c", subcore_axis_name="s")

_f32 = jnp.float32
_i32 = jnp.int32


@functools.partial(
    pl.kernel,
    out_type=jax.ShapeDtypeStruct((2 * N_PAD,), _f32),
    mesh=_mesh,
    scratch_types=[
        pltpu.VMEM_SHARED((N_PAD,), _f32),
        pltpu.VMEM((K,), _i32),
        pltpu.VMEM((K,), _f32),
        pltpu.VMEM((RPT,), _f32),
    ],
)
def _degrees_kernel(src2d, dst2d, dd_o, deg, idx, ones, stage):
    c = lax.axis_index("c")
    s = lax.axis_index("s")

    @pl.loop(0, RPT // 16)
    def _zero(j):
        stage[pl.ds(j * 16, 16)] = jnp.zeros((16,), _f32)

    pltpu.sync_copy(stage, deg.at[pl.ds(s * RPT, RPT)])

    @pl.loop(0, K // 16)
    def _fill(j):
        ones[pl.ds(j * 16, 16)] = jnp.ones((16,), _f32)

    plsc.subcore_barrier()

    @pl.loop(0, C32)
    def _edges(i):
        row = (c * 16 + s) * C32 + i
        pltpu.sync_copy(src2d.at[row], idx)
        pltpu.sync_copy(ones, deg.at[idx], add=True)
        pltpu.sync_copy(dst2d.at[row], idx)
        pltpu.sync_copy(ones, deg.at[idx], add=True)

    plsc.subcore_barrier()

    pltpu.sync_copy(deg.at[pl.ds(s * RPT, RPT)], stage)
    pltpu.sync_copy(stage, dd_o.at[pl.ds(c * N_PAD + s * RPT, RPT)])


@functools.partial(
    pl.kernel,
    out_type=jax.ShapeDtypeStruct((2 * N_PAD, HALF), _f32),
    mesh=_mesh,
    scratch_types=[
        pltpu.VMEM_SHARED((N_PAD, HALF), _f32),
        pltpu.VMEM((K,), _i32),
        pltpu.VMEM((K,), _i32),
        pltpu.VMEM((K, HALF), _f32),
        pltpu.SemaphoreType.DMA,
    ],
    compiler_params=pltpu.CompilerParams(use_tc_tiling_on_sc=False),
)
def _propagate_kernel(zcat, src2d, dstoff2d, s_o, S, didx, sidx, rows, sem):
    c = lax.axis_index("c")
    s = lax.axis_index("s")

    @pl.loop(0, K)
    def _zrow(r):
        rows[r, pl.ds(0, 16)] = jnp.zeros((16,), _f32)
        rows[r, pl.ds(16, 16)] = jnp.zeros((16,), _f32)

    @pl.loop(0, CPY)
    def _zcopy(k):
        pltpu.sync_copy(rows, S.at[pl.ds(s * RPT + k * K, K)])

    plsc.subcore_barrier()

    @pl.loop(0, C16)
    def _edges(i):
        row = s * C16 + i
        pltpu.sync_copy(dstoff2d.at[c * NROW2D + row], didx)
        pltpu.async_copy(zcat.at[didx], rows, sem).wait()
        pltpu.sync_copy(src2d.at[row], sidx)
        pltpu.sync_copy(rows, S.at[sidx], add=True)

    plsc.subcore_barrier()

    @pl.loop(0, CPY)
    def _out(k):
        r0 = s * RPT + k * K
        pltpu.sync_copy(S.at[pl.ds(r0, K)], rows)
        pltpu.sync_copy(rows, s_o.at[pl.ds(c * N_PAD + r0, K)])


def _norm_body(d0_ref, d1_ref, e_ref, w1_ref, w2_ref, z_ref):
    d = d0_ref[...] + d1_ref[...]
    y = jnp.where(d > 0.0, lax.rsqrt(jnp.maximum(d, 1.0)), 0.0)
    w1_ref[...] = jnp.broadcast_to(y, w1_ref.shape)
    w2_ref[...] = jnp.broadcast_to(y * y, w2_ref.shape)
    z_ref[...] = e_ref[...] * y


_norm_kernel = pl.pallas_call(
    _norm_body,
    grid=(2 * N_PAD // BLK,),
    in_specs=[
        pl.BlockSpec((BLK, 1), lambda i: (i % 16, 0)),
        pl.BlockSpec((BLK, 1), lambda i: (i % 16, 0)),
        pl.BlockSpec((BLK, HALF), lambda i: (i, 0)),
    ],
    out_specs=[
        pl.BlockSpec((BLK, HALF), lambda i: (i, 0)),
        pl.BlockSpec((BLK, HALF), lambda i: (i, 0)),
        pl.BlockSpec((BLK, HALF), lambda i: (i, 0)),
    ],
    out_shape=[
        jax.ShapeDtypeStruct((2 * N_PAD, HALF), _f32),
        jax.ShapeDtypeStruct((2 * N_PAD, HALF), _f32),
        jax.ShapeDtypeStruct((2 * N_PAD, HALF), _f32),
    ],
)


def _post_mid_body(s_ref, a_ref, w1_ref, w2_ref, z_ref, ao_ref):
    sv = s_ref[...]
    z_ref[...] = sv * w2_ref[...]
    ao_ref[...] = a_ref[...] + sv * w1_ref[...]


_post_mid = pl.pallas_call(
    _post_mid_body,
    grid=(2 * N_PAD // BLK,),
    in_specs=[pl.BlockSpec((BLK, HALF), lambda i: (i, 0))] * 4,
    out_specs=[pl.BlockSpec((BLK, HALF), lambda i: (i, 0))] * 2,
    out_shape=[
        jax.ShapeDtypeStruct((2 * N_PAD, HALF), _f32),
        jax.ShapeDtypeStruct((2 * N_PAD, HALF), _f32),
    ],
)


def _post_last_body(s_ref, a_ref, w1_ref, ao_ref):
    ao_ref[...] = (a_ref[...] + s_ref[...] * w1_ref[...]) * 0.25


_post_last = pl.pallas_call(
    _post_last_body,
    grid=(2 * N_PAD // BLK,),
    in_specs=[pl.BlockSpec((BLK, HALF), lambda i: (i, 0))] * 3,
    out_specs=pl.BlockSpec((BLK, HALF), lambda i: (i, 0)),
    out_shape=jax.ShapeDtypeStruct((2 * N_PAD, HALF), _f32),
)


def kernel(user_weight, item_weight, edge_index):
    emb0 = jnp.concatenate([user_weight, item_weight], axis=0)
    emb0 = jnp.pad(emb0, ((0, N_PAD - N), (0, 0)))
    ecat = jnp.concatenate([emb0[:, :HALF], emb0[:, HALF:]], axis=0)

    ei = edge_index.astype(jnp.int32)
    src = jnp.pad(ei[0], (0, E_PAD - E), constant_values=TRASH).reshape(-1, K)
    dst = jnp.pad(ei[1], (0, E_PAD - E), constant_values=TRASH).reshape(-1, K)
    dstoff = jnp.concatenate([dst, dst + N_PAD], axis=0)

    dd = _degrees_kernel(src, dst)
    d0 = dd[:N_PAD].reshape(N_PAD, 1)
    d1 = dd[N_PAD:].reshape(N_PAD, 1)
    w1, w2, z = _norm_kernel(d0, d1, ecat)

    acc = ecat
    for _ in range(2):
        s_raw = _propagate_kernel(z, src, dstoff)
        z, acc = _post_mid(s_raw, acc, w1, w2)
    s_raw = _propagate_kernel(z, src, dstoff)
    accf = _post_last(s_raw, acc, w1)

    final = jnp.concatenate([accf[:N], accf[N_PAD:N_PAD + N]], axis=1)
    return (final[:N_USERS], final[N_USERS:], user_weight, item_weight)
